# baseline (speedup 1.0000x reference)
import functools

import jax
import jax.numpy as jnp
from jax.experimental import pallas as pl
from jax.experimental.pallas import tpu as pltpu

_H = 8
_TOPK = 16


def _dot(a, b, dn):
    return jax.lax.dot_general(a, b, dn, preferred_element_type=jnp.float32)


_MM = (((1,), (0,)), ((), ()))
_NT = (((1,), (1,)), ((), ()))


_ROWT = 32


def _threshold_softmax(s_ref, p_ref, k, scale):
    c = s_ref.shape[0]

    def tile(i, _):
        row = i * _ROWT
        st = s_ref[pl.ds(row, _ROWT), :]
        m1 = jnp.max(st, axis=-1, keepdims=True)
        work, m = st, m1
        for _ in range(k - 1):
            work = jnp.where(work == m, -jnp.inf, work)
            m = jnp.max(work, axis=-1, keepdims=True)
        pt = jnp.where(st >= m, jnp.exp((st - m1) * scale), 0.0)
        pt = pt * (1.0 / jnp.sum(pt, axis=-1, keepdims=True))
        p_ref[pl.ds(row, _ROWT), :] = pt
        return 0

    jax.lax.fori_loop(0, c // _ROWT, tile, 0)


def _topk_mask_exact(s, k):
    n = s.shape[-1]
    r = s.shape[0]
    work = s
    cum = jnp.zeros((r, 1), jnp.float32)
    t = jnp.zeros((r, 1), jnp.float32)
    prevc = jnp.zeros((r, 1), jnp.float32)
    kf = jnp.float32(k)
    for _ in range(k):
        m = jnp.max(work, axis=-1, keepdims=True)
        eq = work == m
        cnt = jnp.sum(jnp.where(eq, 1.0, 0.0), axis=-1, keepdims=True)
        newcum = cum + cnt
        crossed = (cum < kf) & (newcum >= kf)
        t = jnp.where(crossed, m, t)
        prevc = jnp.where(crossed, cum, prevc)
        cum = newcum
        work = jnp.where(eq, -jnp.inf, work)
    need = kf - prevc
    eqt = s == t
    ioe = jax.lax.broadcasted_iota(jnp.int32, (n, n), 0)
    ioc = jax.lax.broadcasted_iota(jnp.int32, (n, n), 1)
    ltri = jnp.where(ioe < ioc, 1.0, 0.0)
    rank = _dot(jnp.where(eqt, 1.0, 0.0), ltri, _MM)
    return jnp.where((s > t) | (eqt & (rank < need)), 1.0, 0.0)


def _body(m_ref, wqh_ref, wkh_ref, wvh_ref, wq_ref, wk_ref, woh_ref,
          bo_ref, gate_ref, mt_ref, a_ref, dacc_ref, s_ref, p_ref, *,
          scale):
    h = pl.program_id(1)
    mb = m_ref[0]
    q = _dot(mb, wqh_ref[0], _MM)
    kk = _dot(mb, wkh_ref[0], _MM)
    v = _dot(mb, wvh_ref[0], _MM)
    s_ref[...] = _dot(q, kk, _NT)

    _threshold_softmax(s_ref, p_ref, _TOPK, scale)
    o = _dot(p_ref[...], v, _MM)
    dpart = _dot(o, woh_ref[0], _MM)

    @pl.when(h == 0)
    def _():
        dacc_ref[...] = dpart

    @pl.when(h != 0)
    def _():
        dacc_ref[...] = dacc_ref[...] + dpart

    @pl.when(h == _H - 1)
    def _():
        delta = dacc_ref[...] + bo_ref[...]
        mt_ref[...] = (mb + gate_ref[...] * delta)[None]
        qf = _dot(mb, wq_ref[...], _MM)
        kf = _dot(mb, wk_ref[...], _MM)
        savg = _dot(qf, kf, _NT)
        a_ref[...] = _topk_mask_exact(savg, _TOPK)[None]


def kernel(M, Wq, Wk, Wv, W_out, b_out, gate):
    B, C, NP = M.shape
    DM = Wq.shape[1]
    d = DM // _H
    scale = d ** (-0.5)
    bo = b_out.reshape(1, NP)
    gt = jnp.broadcast_to(jnp.asarray(gate, jnp.float32).reshape(1, 1), (1, NP))
    wq_h = Wq.reshape(NP, _H, d).transpose(1, 0, 2)
    wk_h = Wk.reshape(NP, _H, d).transpose(1, 0, 2)
    wv_h = Wv.reshape(NP, _H, d).transpose(1, 0, 2)
    wo_h = W_out.reshape(_H, d, NP)

    mt, a = pl.pallas_call(
        functools.partial(_body, scale=scale),
        grid=(B, _H),
        in_specs=[
            pl.BlockSpec((1, C, NP), lambda b, h: (b, 0, 0)),
            pl.BlockSpec((1, NP, d), lambda b, h: (h, 0, 0)),
            pl.BlockSpec((1, NP, d), lambda b, h: (h, 0, 0)),
            pl.BlockSpec((1, NP, d), lambda b, h: (h, 0, 0)),
            pl.BlockSpec((NP, DM), lambda b, h: (0, 0)),
            pl.BlockSpec((NP, DM), lambda b, h: (0, 0)),
            pl.BlockSpec((1, d, NP), lambda b, h: (h, 0, 0)),
            pl.BlockSpec((1, NP), lambda b, h: (0, 0)),
            pl.BlockSpec((1, NP), lambda b, h: (0, 0)),
        ],
        out_specs=[
            pl.BlockSpec((1, C, NP), lambda b, h: (b, 0, 0)),
            pl.BlockSpec((1, C, C), lambda b, h: (b, 0, 0)),
        ],
        out_shape=[
            jax.ShapeDtypeStruct((B, C, NP), jnp.float32),
            jax.ShapeDtypeStruct((B, C, C), jnp.float32),
        ],
        scratch_shapes=[
            pltpu.VMEM((C, NP), jnp.float32),
            pltpu.VMEM((C, C), jnp.float32),
            pltpu.VMEM((C, C), jnp.float32),
        ],
        compiler_params=pltpu.CompilerParams(
            dimension_semantics=("parallel", "arbitrary"),
        ),
    )(M, wq_h, wk_h, wv_h, Wq, Wk, wo_h, bo, gt)
    return (mt, a)


# speedup vs baseline: 6.9264x; 6.9264x over previous
import functools

import jax
import jax.numpy as jnp
from jax.experimental import pallas as pl
from jax.experimental.pallas import tpu as pltpu

_H = 8
_TOPK = 16


def _dot(a, b, dn):
    return jax.lax.dot_general(a, b, dn, preferred_element_type=jnp.float32)


_MM = (((1,), (0,)), ((), ()))
_NT = (((1,), (1,)), ((), ()))


_ROWT = 32


def _threshold_softmax(s_ref, p_ref, k, scale):
    c = s_ref.shape[0]

    for i in range(c // _ROWT):
        row = i * _ROWT
        st = s_ref[pl.ds(row, _ROWT), :]
        m1 = jnp.max(st, axis=-1, keepdims=True)
        work, m = st, m1
        for _ in range(k - 1):
            work = jnp.where(work == m, -jnp.inf, work)
            m = jnp.max(work, axis=-1, keepdims=True)
        pt = jnp.where(st >= m, jnp.exp((st - m1) * scale), 0.0)
        pt = pt * (1.0 / jnp.sum(pt, axis=-1, keepdims=True))
        p_ref[pl.ds(row, _ROWT), :] = pt


def _topk_mask_exact(s, k):
    n = s.shape[-1]
    r = s.shape[0]
    work = s
    cum = jnp.zeros((r, 1), jnp.float32)
    t = jnp.zeros((r, 1), jnp.float32)
    prevc = jnp.zeros((r, 1), jnp.float32)
    kf = jnp.float32(k)
    for _ in range(k):
        m = jnp.max(work, axis=-1, keepdims=True)
        eq = work == m
        cnt = jnp.sum(jnp.where(eq, 1.0, 0.0), axis=-1, keepdims=True)
        newcum = cum + cnt
        crossed = (cum < kf) & (newcum >= kf)
        t = jnp.where(crossed, m, t)
        prevc = jnp.where(crossed, cum, prevc)
        cum = newcum
        work = jnp.where(eq, -jnp.inf, work)
    need = kf - prevc
    eqt = s == t
    ioe = jax.lax.broadcasted_iota(jnp.int32, (n, n), 0)
    ioc = jax.lax.broadcasted_iota(jnp.int32, (n, n), 1)
    ltri = jnp.where(ioe < ioc, 1.0, 0.0)
    rank = _dot(jnp.where(eqt, 1.0, 0.0), ltri, _MM)
    return jnp.where((s > t) | (eqt & (rank < need)), 1.0, 0.0)


def _body(m_ref, wqh_ref, wkh_ref, wvh_ref, wq_ref, wk_ref, woh_ref,
          bo_ref, gate_ref, mt_ref, a_ref, dacc_ref, s_ref, p_ref, *,
          scale):
    h = pl.program_id(1)
    mb = m_ref[0]
    q = _dot(mb, wqh_ref[0], _MM)
    kk = _dot(mb, wkh_ref[0], _MM)
    v = _dot(mb, wvh_ref[0], _MM)
    s_ref[...] = _dot(q, kk, _NT)

    _threshold_softmax(s_ref, p_ref, _TOPK, scale)
    o = _dot(p_ref[...], v, _MM)
    dpart = _dot(o, woh_ref[0], _MM)

    @pl.when(h == 0)
    def _():
        dacc_ref[...] = dpart

    @pl.when(h != 0)
    def _():
        dacc_ref[...] = dacc_ref[...] + dpart

    @pl.when(h == _H - 1)
    def _():
        delta = dacc_ref[...] + bo_ref[...]
        mt_ref[...] = (mb + gate_ref[...] * delta)[None]
        qf = _dot(mb, wq_ref[...], _MM)
        kf = _dot(mb, wk_ref[...], _MM)
        savg = _dot(qf, kf, _NT)
        a_ref[...] = _topk_mask_exact(savg, _TOPK)[None]


def kernel(M, Wq, Wk, Wv, W_out, b_out, gate):
    B, C, NP = M.shape
    DM = Wq.shape[1]
    d = DM // _H
    scale = d ** (-0.5)
    bo = b_out.reshape(1, NP)
    gt = jnp.broadcast_to(jnp.asarray(gate, jnp.float32).reshape(1, 1), (1, NP))
    wq_h = Wq.reshape(NP, _H, d).transpose(1, 0, 2)
    wk_h = Wk.reshape(NP, _H, d).transpose(1, 0, 2)
    wv_h = Wv.reshape(NP, _H, d).transpose(1, 0, 2)
    wo_h = W_out.reshape(_H, d, NP)

    mt, a = pl.pallas_call(
        functools.partial(_body, scale=scale),
        grid=(B, _H),
        in_specs=[
            pl.BlockSpec((1, C, NP), lambda b, h: (b, 0, 0)),
            pl.BlockSpec((1, NP, d), lambda b, h: (h, 0, 0)),
            pl.BlockSpec((1, NP, d), lambda b, h: (h, 0, 0)),
            pl.BlockSpec((1, NP, d), lambda b, h: (h, 0, 0)),
            pl.BlockSpec((NP, DM), lambda b, h: (0, 0)),
            pl.BlockSpec((NP, DM), lambda b, h: (0, 0)),
            pl.BlockSpec((1, d, NP), lambda b, h: (h, 0, 0)),
            pl.BlockSpec((1, NP), lambda b, h: (0, 0)),
            pl.BlockSpec((1, NP), lambda b, h: (0, 0)),
        ],
        out_specs=[
            pl.BlockSpec((1, C, NP), lambda b, h: (b, 0, 0)),
            pl.BlockSpec((1, C, C), lambda b, h: (b, 0, 0)),
        ],
        out_shape=[
            jax.ShapeDtypeStruct((B, C, NP), jnp.float32),
            jax.ShapeDtypeStruct((B, C, C), jnp.float32),
        ],
        scratch_shapes=[
            pltpu.VMEM((C, NP), jnp.float32),
            pltpu.VMEM((C, C), jnp.float32),
            pltpu.VMEM((C, C), jnp.float32),
        ],
        compiler_params=pltpu.CompilerParams(
            dimension_semantics=("parallel", "arbitrary"),
        ),
    )(M, wq_h, wk_h, wv_h, Wq, Wk, wo_h, bo, gt)
    return (mt, a)


# speedup vs baseline: 7.1239x; 1.0285x over previous
import functools

import jax
import jax.numpy as jnp
from jax.experimental import pallas as pl
from jax.experimental.pallas import tpu as pltpu

_H = 8
_TOPK = 16


def _dot(a, b, dn):
    return jax.lax.dot_general(a, b, dn, preferred_element_type=jnp.float32)


_MM = (((1,), (0,)), ((), ()))
_NT = (((1,), (1,)), ((), ()))


def _topk_threshold(s, k):
    m1 = jnp.max(s, axis=-1, keepdims=True)
    work, m = s, m1
    for _ in range(k - 1):
        work = jnp.where(work == m, -jnp.inf, work)
        m = jnp.max(work, axis=-1, keepdims=True)
    return m1, m


def _topk_mask_exact(s, k):
    n = s.shape[-1]
    r = s.shape[0]
    work = s
    cum = jnp.zeros((r, 1), jnp.float32)
    t = jnp.zeros((r, 1), jnp.float32)
    prevc = jnp.zeros((r, 1), jnp.float32)
    kf = jnp.float32(k)
    for _ in range(k):
        m = jnp.max(work, axis=-1, keepdims=True)
        eq = work == m
        cnt = jnp.sum(jnp.where(eq, 1.0, 0.0), axis=-1, keepdims=True)
        newcum = cum + cnt
        crossed = (cum < kf) & (newcum >= kf)
        t = jnp.where(crossed, m, t)
        prevc = jnp.where(crossed, cum, prevc)
        cum = newcum
        work = jnp.where(eq, -jnp.inf, work)
    need = kf - prevc
    eqt = s == t
    ioe = jax.lax.broadcasted_iota(jnp.int32, (n, n), 0)
    ioc = jax.lax.broadcasted_iota(jnp.int32, (n, n), 1)
    ltri = jnp.where(ioe < ioc, 1.0, 0.0)
    rank = _dot(jnp.where(eqt, 1.0, 0.0), ltri, _MM)
    return jnp.where((s > t) | (eqt & (rank < need)), 1.0, 0.0)


def _body(m_ref, wqh_ref, wkh_ref, wvh_ref, wq_ref, wk_ref, woh_ref,
          bo_ref, gate_ref, mt_ref, a_ref, dacc_ref, *, scale):
    h = pl.program_id(1)
    mb = m_ref[0]
    q = _dot(mb, wqh_ref[0], _MM)
    kk = _dot(mb, wkh_ref[0], _MM)
    v = _dot(mb, wvh_ref[0], _MM)
    s = _dot(q, kk, _NT)

    m1, t = _topk_threshold(s, _TOPK)
    p = jnp.where(s >= t, jnp.exp((s - m1) * scale), 0.0)
    denom = jnp.sum(p, axis=-1, keepdims=True)
    o = _dot(p, v, _MM) / denom
    dpart = _dot(o, woh_ref[0], _MM)

    @pl.when(h == 0)
    def _():
        dacc_ref[...] = dpart

    @pl.when(h != 0)
    def _():
        dacc_ref[...] = dacc_ref[...] + dpart

    @pl.when(h == _H - 1)
    def _():
        delta = dacc_ref[...] + bo_ref[...]
        mt_ref[...] = (mb + gate_ref[...] * delta)[None]
        qf = _dot(mb, wq_ref[...], _MM)
        kf = _dot(mb, wk_ref[...], _MM)
        savg = _dot(qf, kf, _NT)
        a_ref[...] = _topk_mask_exact(savg, _TOPK)[None]


def kernel(M, Wq, Wk, Wv, W_out, b_out, gate):
    B, C, NP = M.shape
    DM = Wq.shape[1]
    d = DM // _H
    scale = d ** (-0.5)
    bo = b_out.reshape(1, NP)
    gt = jnp.broadcast_to(jnp.asarray(gate, jnp.float32).reshape(1, 1), (1, NP))
    wq_h = Wq.reshape(NP, _H, d).transpose(1, 0, 2)
    wk_h = Wk.reshape(NP, _H, d).transpose(1, 0, 2)
    wv_h = Wv.reshape(NP, _H, d).transpose(1, 0, 2)
    wo_h = W_out.reshape(_H, d, NP)

    mt, a = pl.pallas_call(
        functools.partial(_body, scale=scale),
        grid=(B, _H),
        in_specs=[
            pl.BlockSpec((1, C, NP), lambda b, h: (b, 0, 0)),
            pl.BlockSpec((1, NP, d), lambda b, h: (h, 0, 0)),
            pl.BlockSpec((1, NP, d), lambda b, h: (h, 0, 0)),
            pl.BlockSpec((1, NP, d), lambda b, h: (h, 0, 0)),
            pl.BlockSpec((NP, DM), lambda b, h: (0, 0)),
            pl.BlockSpec((NP, DM), lambda b, h: (0, 0)),
            pl.BlockSpec((1, d, NP), lambda b, h: (h, 0, 0)),
            pl.BlockSpec((1, NP), lambda b, h: (0, 0)),
            pl.BlockSpec((1, NP), lambda b, h: (0, 0)),
        ],
        out_specs=[
            pl.BlockSpec((1, C, NP), lambda b, h: (b, 0, 0)),
            pl.BlockSpec((1, C, C), lambda b, h: (b, 0, 0)),
        ],
        out_shape=[
            jax.ShapeDtypeStruct((B, C, NP), jnp.float32),
            jax.ShapeDtypeStruct((B, C, C), jnp.float32),
        ],
        scratch_shapes=[
            pltpu.VMEM((C, NP), jnp.float32),
        ],
        compiler_params=pltpu.CompilerParams(
            dimension_semantics=("parallel", "arbitrary"),
        ),
    )(M, wq_h, wk_h, wv_h, Wq, Wk, wo_h, bo, gt)
    return (mt, a)


# speedup vs baseline: 8.0789x; 1.1341x over previous
import functools

import jax
import jax.numpy as jnp
from jax import lax
from jax.experimental import pallas as pl
from jax.experimental.pallas import tpu as pltpu
from jax.experimental.pallas import tpu_sc as plsc

_H = 8
_TOPK = 16


def _dot(a, b, dn):
    return jax.lax.dot_general(a, b, dn, preferred_element_type=jnp.float32)


_MM = (((1,), (0,)), ((), ()))
_NT = (((1,), (1,)), ((), ()))


def _savg_body(m_ref, wq_ref, wk_ref, savg_ref):
    mb = m_ref[0]
    qf = _dot(mb, wq_ref[...], _MM)
    kf = _dot(mb, wk_ref[...], _MM)
    savg_ref[...] = _dot(qf, kf, _NT)[None]


_BR = 16


def _sc_row_mask(in_v, out_v, r, c):
    nch = c // 16

    def _sort(x):
        return plsc.sort_key_val(x, x)[0]

    sorted_chunks = [_sort(in_v[r, pl.ds(j * 16, 16)]) for j in range(nch)]
    while len(sorted_chunks) > 1:
        nxt = []
        for a, b in zip(sorted_chunks[::2], sorted_chunks[1::2]):
            top = jnp.maximum(a, lax.rev(b, (0,)))
            nxt.append(_sort(top))
        sorted_chunks = nxt
    t = jnp.min(sorted_chunks[0])
    for j in range(nch):
        ch = in_v[r, pl.ds(j * 16, 16)]
        out_v[r, pl.ds(j * 16, 16)] = jnp.where(ch >= t, 1.0, 0.0)


def _sc_mask(savg2d):
    rows, c = savg2d.shape
    info = plsc.get_sparse_core_info()
    nw = info.num_cores * info.num_subcores
    rows_per_w = rows // nw
    nblk = rows_per_w // _BR
    mesh = plsc.VectorSubcoreMesh(core_axis_name="c", subcore_axis_name="s")

    @functools.partial(
        pl.kernel, mesh=mesh,
        out_type=jax.ShapeDtypeStruct((rows, c), jnp.float32),
        scratch_types=[
            pltpu.VMEM((_BR, c), jnp.float32),
            pltpu.VMEM((_BR, c), jnp.float32),
        ],
        compiler_params=pltpu.CompilerParams(needs_layout_passes=False),
    )
    def k(savg_hbm, out_hbm, in_v, out_v):
        wid = lax.axis_index("s") * info.num_cores + lax.axis_index("c")
        base = wid * rows_per_w

        def blk(i, carry):
            row0 = base + i * _BR
            pltpu.sync_copy(savg_hbm.at[pl.ds(row0, _BR)], in_v)
            for r in range(_BR):
                _sc_row_mask(in_v, out_v, r, c)
            pltpu.sync_copy(out_v, out_hbm.at[pl.ds(row0, _BR)])
            return carry

        lax.fori_loop(0, nblk, blk, 0)

    return k(savg2d)


def _topk_threshold(s, k):
    m1 = jnp.max(s, axis=-1, keepdims=True)
    work, m = s, m1
    for _ in range(k - 1):
        work = jnp.where(work == m, -jnp.inf, work)
        m = jnp.max(work, axis=-1, keepdims=True)
    return m1, m


def _attn_body(m_ref, wqh_ref, wkh_ref, wvh_ref, woh_ref,
               bo_ref, gate_ref, mt_ref, dacc_ref, *, scale):
    h = pl.program_id(1)
    mb = m_ref[0]
    q = _dot(mb, wqh_ref[0], _MM)
    kk = _dot(mb, wkh_ref[0], _MM)
    v = _dot(mb, wvh_ref[0], _MM)
    s = _dot(q, kk, _NT)

    m1, t = _topk_threshold(s, _TOPK)
    p = jnp.where(s >= t, jnp.exp((s - m1) * scale), 0.0)
    denom = jnp.sum(p, axis=-1, keepdims=True)
    o = _dot(p, v, _MM) / denom
    dpart = _dot(o, woh_ref[0], _MM)

    @pl.when(h == 0)
    def _():
        dacc_ref[...] = dpart

    @pl.when(h != 0)
    def _():
        dacc_ref[...] = dacc_ref[...] + dpart

    @pl.when(h == _H - 1)
    def _():
        delta = dacc_ref[...] + bo_ref[...]
        mt_ref[...] = (mb + gate_ref[...] * delta)[None]


def kernel(M, Wq, Wk, Wv, W_out, b_out, gate):
    B, C, NP = M.shape
    DM = Wq.shape[1]
    d = DM // _H
    scale = d ** (-0.5)
    bo = b_out.reshape(1, NP)
    gt = jnp.broadcast_to(jnp.asarray(gate, jnp.float32).reshape(1, 1), (1, NP))
    wq_h = Wq.reshape(NP, _H, d).transpose(1, 0, 2)
    wk_h = Wk.reshape(NP, _H, d).transpose(1, 0, 2)
    wv_h = Wv.reshape(NP, _H, d).transpose(1, 0, 2)
    wo_h = W_out.reshape(_H, d, NP)

    savg = pl.pallas_call(
        _savg_body,
        grid=(B,),
        in_specs=[
            pl.BlockSpec((1, C, NP), lambda b: (b, 0, 0)),
            pl.BlockSpec((NP, DM), lambda b: (0, 0)),
            pl.BlockSpec((NP, DM), lambda b: (0, 0)),
        ],
        out_specs=pl.BlockSpec((1, C, C), lambda b: (b, 0, 0)),
        out_shape=jax.ShapeDtypeStruct((B, C, C), jnp.float32),
        compiler_params=pltpu.CompilerParams(
            dimension_semantics=("parallel",),
        ),
    )(M, Wq, Wk)

    a = _sc_mask(savg.reshape(B * C, C)).reshape(B, C, C)

    mt = pl.pallas_call(
        functools.partial(_attn_body, scale=scale),
        grid=(B, _H),
        in_specs=[
            pl.BlockSpec((1, C, NP), lambda b, h: (b, 0, 0)),
            pl.BlockSpec((1, NP, d), lambda b, h: (h, 0, 0)),
            pl.BlockSpec((1, NP, d), lambda b, h: (h, 0, 0)),
            pl.BlockSpec((1, NP, d), lambda b, h: (h, 0, 0)),
            pl.BlockSpec((1, d, NP), lambda b, h: (h, 0, 0)),
            pl.BlockSpec((1, NP), lambda b, h: (0, 0)),
            pl.BlockSpec((1, NP), lambda b, h: (0, 0)),
        ],
        out_specs=pl.BlockSpec((1, C, NP), lambda b, h: (b, 0, 0)),
        out_shape=jax.ShapeDtypeStruct((B, C, NP), jnp.float32),
        scratch_shapes=[
            pltpu.VMEM((C, NP), jnp.float32),
        ],
        compiler_params=pltpu.CompilerParams(
            dimension_semantics=("parallel", "arbitrary"),
        ),
    )(M, wq_h, wk_h, wv_h, wo_h, bo, gt)
    return (mt, a)


# speedup vs baseline: 8.1060x; 1.0033x over previous
import functools

import jax
import jax.numpy as jnp
from jax import lax
from jax.experimental import pallas as pl
from jax.experimental.pallas import tpu as pltpu
from jax.experimental.pallas import tpu_sc as plsc

_H = 8
_TOPK = 16


def _dot(a, b, dn):
    return jax.lax.dot_general(a, b, dn, preferred_element_type=jnp.float32)


_MM = (((1,), (0,)), ((), ()))
_NT = (((1,), (1,)), ((), ()))


def _savg_body(m_ref, wq_ref, wk_ref, savg_ref):
    mb = m_ref[0]
    qf = _dot(mb, wq_ref[...], _MM)
    kf = _dot(mb, wk_ref[...], _MM)
    savg_ref[...] = _dot(qf, kf, _NT)[None]


_BR = 16


def _sc_row_mask(in_v, out_v, r, c):
    nch = c // 16

    def _sort(x):
        return plsc.sort_key_val(x, x)[0]

    sorted_chunks = [_sort(in_v[r, pl.ds(j * 16, 16)]) for j in range(nch)]
    while len(sorted_chunks) > 1:
        nxt = []
        for a, b in zip(sorted_chunks[::2], sorted_chunks[1::2]):
            top = jnp.maximum(a, lax.rev(b, (0,)))
            nxt.append(_sort(top))
        sorted_chunks = nxt
    t = jnp.min(sorted_chunks[0])
    for j in range(nch):
        ch = in_v[r, pl.ds(j * 16, 16)]
        out_v[r, pl.ds(j * 16, 16)] = jnp.where(ch >= t, 1.0, 0.0)


def _sc_mask(savg2d):
    rows, c = savg2d.shape
    info = plsc.get_sparse_core_info()
    nw = info.num_cores * info.num_subcores
    rows_per_w = rows // nw
    nblk = rows_per_w // _BR
    mesh = plsc.VectorSubcoreMesh(core_axis_name="c", subcore_axis_name="s")

    @functools.partial(
        pl.kernel, mesh=mesh,
        out_type=jax.ShapeDtypeStruct((rows, c), jnp.float32),
        scratch_types=[
            pltpu.VMEM((_BR, c), jnp.float32),
            pltpu.VMEM((_BR, c), jnp.float32),
        ],
        compiler_params=pltpu.CompilerParams(needs_layout_passes=False),
    )
    def k(savg_hbm, out_hbm, in_v, out_v):
        wid = lax.axis_index("s") * info.num_cores + lax.axis_index("c")
        base = wid * rows_per_w

        def blk(i, carry):
            row0 = base + i * _BR
            pltpu.sync_copy(savg_hbm.at[pl.ds(row0, _BR)], in_v)
            for r in range(_BR):
                _sc_row_mask(in_v, out_v, r, c)
            pltpu.sync_copy(out_v, out_hbm.at[pl.ds(row0, _BR)])
            return carry

        lax.fori_loop(0, nblk, blk, 0)

    return k(savg2d)


def _topk_threshold(s, k):
    m1 = jnp.max(s, axis=-1, keepdims=True)
    m = m1
    for _ in range(k - 1):
        m = jnp.max(jnp.where(s < m, s, -jnp.inf), axis=-1, keepdims=True)
    return m1, m


def _attn_body(m_ref, wqh_ref, wkh_ref, wvh_ref, woh_ref,
               bo_ref, gate_ref, mt_ref, dacc_ref, *, scale):
    h = pl.program_id(1)
    mb = m_ref[0]
    q = _dot(mb, wqh_ref[0], _MM)
    kk = _dot(mb, wkh_ref[0], _MM)
    v = _dot(mb, wvh_ref[0], _MM)
    s = _dot(q, kk, _NT)

    m1, t = _topk_threshold(s, _TOPK)
    p = jnp.where(s >= t, jnp.exp((s - m1) * scale), 0.0)
    denom = jnp.sum(p, axis=-1, keepdims=True)
    o = _dot(p, v, _MM) / denom
    dpart = _dot(o, woh_ref[0], _MM)

    @pl.when(h == 0)
    def _():
        dacc_ref[...] = dpart

    @pl.when(h != 0)
    def _():
        dacc_ref[...] = dacc_ref[...] + dpart

    @pl.when(h == _H - 1)
    def _():
        delta = dacc_ref[...] + bo_ref[...]
        mt_ref[...] = (mb + gate_ref[...] * delta)[None]


def kernel(M, Wq, Wk, Wv, W_out, b_out, gate):
    B, C, NP = M.shape
    DM = Wq.shape[1]
    d = DM // _H
    scale = d ** (-0.5)
    bo = b_out.reshape(1, NP)
    gt = jnp.broadcast_to(jnp.asarray(gate, jnp.float32).reshape(1, 1), (1, NP))
    wq_h = Wq.reshape(NP, _H, d).transpose(1, 0, 2)
    wk_h = Wk.reshape(NP, _H, d).transpose(1, 0, 2)
    wv_h = Wv.reshape(NP, _H, d).transpose(1, 0, 2)
    wo_h = W_out.reshape(_H, d, NP)

    savg = pl.pallas_call(
        _savg_body,
        grid=(B,),
        in_specs=[
            pl.BlockSpec((1, C, NP), lambda b: (b, 0, 0)),
            pl.BlockSpec((NP, DM), lambda b: (0, 0)),
            pl.BlockSpec((NP, DM), lambda b: (0, 0)),
        ],
        out_specs=pl.BlockSpec((1, C, C), lambda b: (b, 0, 0)),
        out_shape=jax.ShapeDtypeStruct((B, C, C), jnp.float32),
        compiler_params=pltpu.CompilerParams(
            dimension_semantics=("parallel",),
        ),
    )(M, Wq, Wk)

    a = _sc_mask(savg.reshape(B * C, C)).reshape(B, C, C)

    mt = pl.pallas_call(
        functools.partial(_attn_body, scale=scale),
        grid=(B, _H),
        in_specs=[
            pl.BlockSpec((1, C, NP), lambda b, h: (b, 0, 0)),
            pl.BlockSpec((1, NP, d), lambda b, h: (h, 0, 0)),
            pl.BlockSpec((1, NP, d), lambda b, h: (h, 0, 0)),
            pl.BlockSpec((1, NP, d), lambda b, h: (h, 0, 0)),
            pl.BlockSpec((1, d, NP), lambda b, h: (h, 0, 0)),
            pl.BlockSpec((1, NP), lambda b, h: (0, 0)),
            pl.BlockSpec((1, NP), lambda b, h: (0, 0)),
        ],
        out_specs=pl.BlockSpec((1, C, NP), lambda b, h: (b, 0, 0)),
        out_shape=jax.ShapeDtypeStruct((B, C, NP), jnp.float32),
        scratch_shapes=[
            pltpu.VMEM((C, NP), jnp.float32),
        ],
        compiler_params=pltpu.CompilerParams(
            dimension_semantics=("parallel", "arbitrary"),
        ),
    )(M, wq_h, wk_h, wv_h, wo_h, bo, gt)
    return (mt, a)


# speedup vs baseline: 8.2752x; 1.0209x over previous
import functools

import jax
import jax.numpy as jnp
from jax import lax
from jax.experimental import pallas as pl
from jax.experimental.pallas import tpu as pltpu
from jax.experimental.pallas import tpu_sc as plsc

_H = 8
_TOPK = 16


def _dot(a, b, dn):
    return jax.lax.dot_general(a, b, dn, preferred_element_type=jnp.float32)


_MM = (((1,), (0,)), ((), ()))
_NT = (((1,), (1,)), ((), ()))


def _savg_body(m_ref, wq_ref, wk_ref, savg_ref):
    mb = m_ref[0]
    qf = _dot(mb, wq_ref[...], _MM)
    kf = _dot(mb, wk_ref[...], _MM)
    savg_ref[...] = _dot(qf, kf, _NT)[None]


_BR = 16


def _sc_row_mask(in_v, out_v, r, c):
    nch = c // 16

    def _sort(x):
        return plsc.sort_key_val(x, x)[0]

    sorted_chunks = [_sort(in_v[r, pl.ds(j * 16, 16)]) for j in range(nch)]
    while len(sorted_chunks) > 1:
        nxt = []
        for a, b in zip(sorted_chunks[::2], sorted_chunks[1::2]):
            top = jnp.maximum(a, lax.rev(b, (0,)))
            nxt.append(_sort(top))
        sorted_chunks = nxt
    t = jnp.min(sorted_chunks[0])
    for j in range(nch):
        ch = in_v[r, pl.ds(j * 16, 16)]
        out_v[r, pl.ds(j * 16, 16)] = jnp.where(ch >= t, 1.0, 0.0)


def _sc_mask(savg2d):
    rows, c = savg2d.shape
    info = plsc.get_sparse_core_info()
    nw = info.num_cores * info.num_subcores
    rows_per_w = rows // nw
    nblk = rows_per_w // _BR
    mesh = plsc.VectorSubcoreMesh(core_axis_name="c", subcore_axis_name="s")

    @functools.partial(
        pl.kernel, mesh=mesh,
        out_type=jax.ShapeDtypeStruct((rows, c), jnp.float32),
        scratch_types=[
            pltpu.VMEM((_BR, c), jnp.float32),
            pltpu.VMEM((_BR, c), jnp.float32),
        ],
        compiler_params=pltpu.CompilerParams(needs_layout_passes=False),
    )
    def k(savg_hbm, out_hbm, in_v, out_v):
        wid = lax.axis_index("s") * info.num_cores + lax.axis_index("c")
        base = wid * rows_per_w

        def blk(i, carry):
            row0 = base + i * _BR
            pltpu.sync_copy(savg_hbm.at[pl.ds(row0, _BR)], in_v)
            for r in range(_BR):
                _sc_row_mask(in_v, out_v, r, c)
            pltpu.sync_copy(out_v, out_hbm.at[pl.ds(row0, _BR)])
            return carry

        lax.fori_loop(0, nblk, blk, 0)

    return k(savg2d)


def _topk_threshold(s, k):
    m1 = jnp.max(s, axis=-1, keepdims=True)
    m = m1
    for _ in range(k - 1):
        m = jnp.max(jnp.where(s < m, s, -jnp.inf), axis=-1, keepdims=True)
    return m1, m


def _attn_body(m_ref, wqh_ref, wkh_ref, wvh_ref, woh_ref,
               bo_ref, gate_ref, mt_ref, dacc_ref, *, scale):
    h = pl.program_id(1)
    mb = m_ref[0]
    q = _dot(mb, wqh_ref[0], _MM)
    kk = _dot(mb, wkh_ref[0], _MM)
    v = _dot(mb, wvh_ref[0], _MM)
    s = _dot(q, kk, _NT)
    vw = _dot(v, woh_ref[0], _MM)

    m1, t = _topk_threshold(s, _TOPK)
    p = jnp.where(s >= t, jnp.exp((s - m1) * scale), 0.0)
    denom = jnp.sum(p, axis=-1, keepdims=True)
    dpart = _dot(p, vw, _MM) / denom

    @pl.when(h == 0)
    def _():
        dacc_ref[...] = dpart

    @pl.when(h != 0)
    def _():
        dacc_ref[...] = dacc_ref[...] + dpart

    @pl.when(h == _H - 1)
    def _():
        delta = dacc_ref[...] + bo_ref[...]
        mt_ref[...] = (mb + gate_ref[...] * delta)[None]


def kernel(M, Wq, Wk, Wv, W_out, b_out, gate):
    B, C, NP = M.shape
    DM = Wq.shape[1]
    d = DM // _H
    scale = d ** (-0.5)
    bo = b_out.reshape(1, NP)
    gt = jnp.broadcast_to(jnp.asarray(gate, jnp.float32).reshape(1, 1), (1, NP))
    wq_h = Wq.reshape(NP, _H, d).transpose(1, 0, 2)
    wk_h = Wk.reshape(NP, _H, d).transpose(1, 0, 2)
    wv_h = Wv.reshape(NP, _H, d).transpose(1, 0, 2)
    wo_h = W_out.reshape(_H, d, NP)

    savg = pl.pallas_call(
        _savg_body,
        grid=(B,),
        in_specs=[
            pl.BlockSpec((1, C, NP), lambda b: (b, 0, 0)),
            pl.BlockSpec((NP, DM), lambda b: (0, 0)),
            pl.BlockSpec((NP, DM), lambda b: (0, 0)),
        ],
        out_specs=pl.BlockSpec((1, C, C), lambda b: (b, 0, 0)),
        out_shape=jax.ShapeDtypeStruct((B, C, C), jnp.float32),
        compiler_params=pltpu.CompilerParams(
            dimension_semantics=("parallel",),
        ),
    )(M, Wq, Wk)

    a = _sc_mask(savg.reshape(B * C, C)).reshape(B, C, C)

    mt = pl.pallas_call(
        functools.partial(_attn_body, scale=scale),
        grid=(B, _H),
        in_specs=[
            pl.BlockSpec((1, C, NP), lambda b, h: (b, 0, 0)),
            pl.BlockSpec((1, NP, d), lambda b, h: (h, 0, 0)),
            pl.BlockSpec((1, NP, d), lambda b, h: (h, 0, 0)),
            pl.BlockSpec((1, NP, d), lambda b, h: (h, 0, 0)),
            pl.BlockSpec((1, d, NP), lambda b, h: (h, 0, 0)),
            pl.BlockSpec((1, NP), lambda b, h: (0, 0)),
            pl.BlockSpec((1, NP), lambda b, h: (0, 0)),
        ],
        out_specs=pl.BlockSpec((1, C, NP), lambda b, h: (b, 0, 0)),
        out_shape=jax.ShapeDtypeStruct((B, C, NP), jnp.float32),
        scratch_shapes=[
            pltpu.VMEM((C, NP), jnp.float32),
        ],
        compiler_params=pltpu.CompilerParams(
            dimension_semantics=("parallel", "arbitrary"),
        ),
    )(M, wq_h, wk_h, wv_h, wo_h, bo, gt)
    return (mt, a)
